# baseline (speedup 1.0000x reference)
import functools

import jax
import jax.numpy as jnp
from jax import lax
from jax.experimental import pallas as pl
from jax.experimental.pallas import tpu as pltpu

_TOPK = 0.25


def _topk_mask_kernel(x_ref, o_ref, bits_ref, *, k):
    x = x_ref[0]
    bits_ref[...] = lax.bitcast_convert_type(jnp.abs(x), jnp.int32)
    p = x.shape[1]
    lo0 = jnp.zeros((1, p), jnp.int32)
    hi0 = jnp.full((1, p), jnp.int32(0x7FFFFFFF), jnp.int32)

    c_dim = x.shape[0]

    def body(i, c):
        lo, hi = c
        mid = lo + ((hi - lo) >> 1)
        lt = lax.shift_right_logical(bits_ref[...] - mid, 31)
        cnt_lt = jnp.sum(lt, axis=0, keepdims=True)
        ge = cnt_lt <= (c_dim - k)
        return jnp.where(ge, mid, lo), jnp.where(ge, hi, mid)

    lo, _ = lax.fori_loop(0, 31, body, (lo0, hi0))
    o_ref[0] = jnp.where(bits_ref[...] >= lo, x, jnp.zeros_like(x))


def kernel(x, tau):
    n, c, h, w = x.shape
    k = max(int(_TOPK * c), 1)
    p = h * w
    xr = x.reshape(n, c, p)
    sparse = pl.pallas_call(
        functools.partial(_topk_mask_kernel, k=k),
        out_shape=jax.ShapeDtypeStruct((n, c, p), x.dtype),
        grid=(n,),
        in_specs=[pl.BlockSpec((1, c, p), lambda i: (i, 0, 0))],
        out_specs=pl.BlockSpec((1, c, p), lambda i: (i, 0, 0)),
        scratch_shapes=[pltpu.VMEM((c, p), jnp.int32)],
    )(xr).reshape(n, c, h, w)
    tau_arr = jnp.asarray(tau)
    tau_f = tau_arr.astype(x.dtype)
    blended = sparse * tau_f + x * (1.0 - tau_f)
    return jnp.where(tau_arr == 1, sparse, blended)


# speedup vs baseline: 1.0692x; 1.0692x over previous
import functools

import jax
import jax.numpy as jnp
from jax import lax
from jax.experimental import pallas as pl
from jax.experimental.pallas import tpu as pltpu

_TOPK = 0.25


def _topk_mask_kernel(x_ref, o_ref, bits_ref, *, k):
    x = x_ref[0]
    bits_ref[...] = lax.bitcast_convert_type(jnp.abs(x), jnp.int32)
    p = x.shape[1]
    lo0 = jnp.zeros((1, p), jnp.int32)
    hi0 = jnp.full((1, p), jnp.int32(0x7FFFFFFF), jnp.int32)

    ones_row = jnp.ones((1, x.shape[0]), jnp.float32)

    def body(i, c):
        lo, hi = c
        mid = lo + ((hi - lo) >> 1)
        ind = jnp.where(bits_ref[...] >= mid, 1.0, 0.0)
        cnt = jax.lax.dot_general(
            ones_row, ind, (((1,), (0,)), ((), ())),
            preferred_element_type=jnp.float32)
        ge = cnt >= jnp.float32(k)
        return jnp.where(ge, mid, lo), jnp.where(ge, hi, mid)

    lo, _ = lax.fori_loop(0, 31, body, (lo0, hi0))
    o_ref[0] = jnp.where(bits_ref[...] >= lo, x, jnp.zeros_like(x))


def kernel(x, tau):
    n, c, h, w = x.shape
    k = max(int(_TOPK * c), 1)
    p = h * w
    xr = x.reshape(n, c, p)
    sparse = pl.pallas_call(
        functools.partial(_topk_mask_kernel, k=k),
        out_shape=jax.ShapeDtypeStruct((n, c, p), x.dtype),
        grid=(n,),
        in_specs=[pl.BlockSpec((1, c, p), lambda i: (i, 0, 0))],
        out_specs=pl.BlockSpec((1, c, p), lambda i: (i, 0, 0)),
        scratch_shapes=[pltpu.VMEM((c, p), jnp.int32)],
    )(xr).reshape(n, c, h, w)
    tau_arr = jnp.asarray(tau)
    tau_f = tau_arr.astype(x.dtype)
    blended = sparse * tau_f + x * (1.0 - tau_f)
    return jnp.where(tau_arr == 1, sparse, blended)
